# baseline (speedup 1.0000x reference)
import jax
import jax.numpy as jnp
from jax.experimental import pallas as pl

N_NODES = 10000
K = 16


def _leaky(h, s):
    return jnp.where(h >= 0, h, s * h)


def _bn(h, g, b):
    m = jnp.mean(h, axis=0)
    v = jnp.var(h, axis=0)
    return g * (h - m) / jnp.sqrt(v + 1e-5) + b


def _mlp(h, layers, slope):
    for (W, g, b) in layers:
        h = _leaky(_bn(h @ W.T, g, b), slope)
    return h


def _seg_max(vals, idx, n):
    out = jax.ops.segment_max(vals, idx, num_segments=n)
    return jnp.where(jnp.isfinite(out), out, 0.0)


def _static_conv(x, edge_index, edge_attr, layers, n):
    src = edge_index[0]
    dst = edge_index[1]
    x_i = x[dst]
    x_j = x[src]
    msg = jnp.concatenate([x_i, x_j - x_i, edge_attr], axis=1)
    agg = _seg_max(msg, dst, n)
    return _mlp(agg, layers, 0.1)


def _dynamic_conv(x, k, layers, n):
    xs = jax.lax.stop_gradient(x)
    xn = xs / (jnp.linalg.norm(xs, axis=1, keepdims=True) + 1e-8)
    sim = xn @ xn.T
    _, nbr = jax.lax.top_k(sim, K)
    k_res = jnp.asarray(k, jnp.int32) - K
    dst = jnp.repeat(jnp.arange(n), K) + k_res
    src = nbr.reshape(-1)
    x_i = x[dst]
    x_j = x[src]
    msg = jnp.concatenate([x_i, x_j - x_i], axis=1)
    msg = _mlp(msg, layers, 0.2)
    return _seg_max(msg, dst, n)


def _final_matmul_kernel(h_ref, w_ref, o_ref):
    o_ref[...] = jnp.dot(h_ref[...], w_ref[...],
                         preferred_element_type=jnp.float32)


def kernel(x, edge_index, edge_attr, params, k):
    n = x.shape[0]
    sg1 = _static_conv(x, edge_index, edge_attr, params['sg1'], n)
    sg2 = _static_conv(sg1, edge_index, edge_attr, params['sg2'], n)
    sg3 = _static_conv(sg2, edge_index, edge_attr, params['sg3'], n)
    dg1 = _dynamic_conv(sg1, k, params['dg1'], n)
    dg2 = _dynamic_conv(dg1, k, params['dg2'], n)
    cat1 = jnp.concatenate([sg1, dg1, dg2, sg2, sg3], axis=1)
    f1 = _mlp(cat1, params['fuse1'], 0.2)
    cat2 = jnp.concatenate([f1, cat1], axis=1)
    h = _mlp(cat2, params['fuse2'], 0.2)
    wt = params['fuse2_out'].T
    out = pl.pallas_call(
        _final_matmul_kernel,
        out_shape=jax.ShapeDtypeStruct((n, wt.shape[1]), jnp.float32),
        grid=(10,),
        in_specs=[pl.BlockSpec((n // 10, wt.shape[0]), lambda i: (i, 0)),
                  pl.BlockSpec((wt.shape[0], wt.shape[1]), lambda i: (0, 0))],
        out_specs=pl.BlockSpec((n // 10, wt.shape[1]), lambda i: (i, 0)),
    )(h, wt)
    return out


# speedup vs baseline: 1.2260x; 1.2260x over previous
import functools

import jax
import jax.numpy as jnp
from jax import lax
from jax.experimental import pallas as pl


def _pick_bm(m, cap=1600):
    for c in (1600, 1000, 800, 400, 320, 200, 160, 100, 80, 40, 16, 8, 4, 2, 1):
        if c <= cap and m % c == 0:
            return c
    return m


def _mm_body(x_ref, m_ref, g_ref, d_ref, b_ref, sl_ref, w_ref, y_ref, st_ref):
    i = pl.program_id(1)
    z = g_ref[...] * (x_ref[...] - m_ref[...]) / d_ref[...] + b_ref[...]
    z = jnp.where(z >= 0, z, z * sl_ref[...])
    y = jnp.dot(z.astype(jnp.bfloat16), w_ref[...],
                preferred_element_type=jnp.float32)
    y_ref[...] = y

    @pl.when(i == 0)
    def _():
        st_ref[...] = jnp.zeros_like(st_ref)

    s = jnp.sum(y, axis=0, keepdims=True)
    ss = jnp.sum(y * y, axis=0, keepdims=True)
    st_ref[...] += jnp.concatenate([s, ss], axis=0)


def _fused_mm(x, bn, sl, wt):
    m, kd = x.shape
    nd = wt.shape[1]
    bm = _pick_bm(m)
    if nd <= 1024:
        bn_blk = nd
    else:
        assert nd % 1024 == 0, nd
        bn_blk = 1024
    grid = (nd // bn_blk, m // bm)
    wt = wt.astype(jnp.bfloat16)
    row = pl.BlockSpec((1, kd), lambda j, i: (0, 0))
    y, st = pl.pallas_call(
        _mm_body,
        grid=grid,
        in_specs=[
            pl.BlockSpec((bm, kd), lambda j, i: (i, 0)),
            row, row, row, row, row,
            pl.BlockSpec((kd, bn_blk), lambda j, i: (0, j)),
        ],
        out_specs=[
            pl.BlockSpec((bm, bn_blk), lambda j, i: (i, j)),
            pl.BlockSpec((2, bn_blk), lambda j, i: (0, j)),
        ],
        out_shape=[
            jax.ShapeDtypeStruct((m, nd), jnp.float32),
            jax.ShapeDtypeStruct((2, nd), jnp.float32),
        ],
    )(x, bn[0], bn[1], bn[2], bn[3], sl, wt)
    return y, st


def _bn_id(kd):
    z = jnp.zeros((1, kd), jnp.float32)
    o = jnp.ones((1, kd), jnp.float32)
    return (z, o, o, z)


def _ones_row(kd):
    return jnp.ones((1, kd), jnp.float32)


def _fin_stats(st, g, b, m):
    mean = st[0] / m
    var = jnp.maximum(st[1] / m - mean * mean, 0.0)
    den = jnp.sqrt(var + 1e-5)
    return (mean[None], g[None], den[None], b[None])


def _fin_stats_exact(y, g, b):
    mean = jnp.mean(y, axis=0)
    den = jnp.sqrt(jnp.var(y, axis=0) + 1e-5)
    return (mean[None], g[None], den[None], b[None])


def _act_body(y_ref, m_ref, g_ref, d_ref, b_ref, o_ref, *, slope):
    z = g_ref[...] * (y_ref[...] - m_ref[...]) / d_ref[...] + b_ref[...]
    o_ref[...] = jnp.where(z >= 0, z, z * slope)


def _apply_act(y, bn, slope):
    m, nd = y.shape
    bm = _pick_bm(m)
    row = pl.BlockSpec((1, nd), lambda i: (0, 0))
    return pl.pallas_call(
        functools.partial(_act_body, slope=slope),
        grid=(m // bm,),
        in_specs=[pl.BlockSpec((bm, nd), lambda i: (i, 0)),
                  row, row, row, row],
        out_specs=pl.BlockSpec((bm, nd), lambda i: (i, 0)),
        out_shape=jax.ShapeDtypeStruct((m, nd), jnp.float32),
    )(y, bn[0], bn[1], bn[2], bn[3])


def _addstats_body(p_ref, q_ref, v_ref, st_ref):
    i = pl.program_id(0)
    v = q_ref[...] + p_ref[...][:, None, :]
    v_ref[...] = v

    @pl.when(i == 0)
    def _():
        st_ref[...] = jnp.zeros_like(st_ref)

    s = jnp.sum(jnp.sum(v, axis=1), axis=0, keepdims=True)
    ss = jnp.sum(jnp.sum(v * v, axis=1), axis=0, keepdims=True)
    st_ref[...] += jnp.concatenate([s, ss], axis=0)


def _edge_lin1(p, qg):
    n, d = p.shape
    kk = qg.shape[1]
    bm = _pick_bm(n, cap=200)
    v, st = pl.pallas_call(
        _addstats_body,
        grid=(n // bm,),
        in_specs=[
            pl.BlockSpec((bm, d), lambda i: (i, 0)),
            pl.BlockSpec((bm, kk, d), lambda i: (i, 0, 0)),
        ],
        out_specs=[
            pl.BlockSpec((bm, kk, d), lambda i: (i, 0, 0)),
            pl.BlockSpec((2, d), lambda i: (0, 0)),
        ],
        out_shape=[
            jax.ShapeDtypeStruct((n, kk, d), jnp.float32),
            jax.ShapeDtypeStruct((2, d), jnp.float32),
        ],
    )(p, qg)
    return v, st


def _segmax_body(y_ref, o_ref):
    o_ref[...] = jnp.max(y_ref[...], axis=1)


def _segmax_k(y3):
    n, kk, d = y3.shape
    bm = _pick_bm(n, cap=200)
    return pl.pallas_call(
        _segmax_body,
        grid=(n // bm,),
        in_specs=[pl.BlockSpec((bm, kk, d), lambda i: (i, 0, 0))],
        out_specs=pl.BlockSpec((bm, d), lambda i: (i, 0)),
        out_shape=jax.ShapeDtypeStruct((n, d), jnp.float32),
    )(y3)


def _leaky_j(h, s):
    return jnp.where(h >= 0, h, s * h)


def _bn_j(h, g, b):
    m = jnp.mean(h, axis=0)
    v = jnp.var(h, axis=0)
    return g * (h - m) / jnp.sqrt(v + 1e-5) + b


def _mlp_j(h, layers, slope):
    for (w, g, b) in layers:
        h = _leaky_j(_bn_j(h @ w.T, g, b), slope)
    return h


def _mlp_raw(y, st, layers, slope, m, exact=False):
    if exact:
        bn = _fin_stats_exact(y, layers[0][1], layers[0][2])
    else:
        bn = _fin_stats(st, layers[0][1], layers[0][2], m)
    for li in range(1, len(layers)):
        wt = layers[li][0].T
        y, st = _fused_mm(y, bn, jnp.full((1, y.shape[1]), slope), wt)
        if exact:
            bn = _fin_stats_exact(y, layers[li][1], layers[li][2])
        else:
            bn = _fin_stats(st, layers[li][1], layers[li][2], m)
    return y, bn


def _static_agg(xa, src, dst, ea_blk, n):
    smax = jax.ops.segment_max(xa[src], dst, num_segments=n)
    mask = jnp.isfinite(smax[:, :1])
    return jnp.concatenate(
        [
            jnp.where(mask, xa, 0.0),
            jnp.where(mask, smax - xa, 0.0),
            ea_blk,
        ],
        axis=1,
    )


def _static_conv(xa, src, dst, ea_blk, layers, n):
    agg = _static_agg(xa, src, dst, ea_blk, n)
    kd = agg.shape[1]
    y, st = _fused_mm(agg, _bn_id(kd), _ones_row(kd), layers[0][0].T)
    return _mlp_raw(y, st, layers, 0.1, n)


def _knn(xa, n, kk):
    xn = xa / (jnp.linalg.norm(xa, axis=1, keepdims=True) + 1e-8)
    d = xn.shape[1]
    npad = ((n + 1023) // 1024) * 1024
    wt = jnp.pad(xn.T, ((0, 0), (0, npad - n)))
    sim, _ = _fused_mm(xn, _bn_id(d), _ones_row(d), wt)
    _, nbr = lax.top_k(sim[:, :n], kk)
    return nbr


def _dynamic_conv_exact(xa, layers, n, kk):
    nbr = _knn(xa, n, kk)
    x_j = xa[nbr.reshape(-1)]
    x_i = jnp.repeat(xa, kk, axis=0)
    msg = jnp.concatenate([x_i, x_j - x_i], axis=1)
    msg = _mlp_j(msg, layers, 0.2)
    r = jnp.max(msg.reshape(n, kk, msg.shape[1]), axis=1)
    return jnp.where(jnp.isfinite(r), r, 0.0)


def _dynamic_conv(xa, layers, n, kk):
    nbr = _knn(xa, n, kk)
    d = xa.shape[1]
    w1 = layers[0][0]
    h = w1.shape[1] // 2
    w1a, w1b = w1[:, :h], w1[:, h:]
    ab = jnp.concatenate([(w1a - w1b).T, w1b.T], axis=1)
    pq, _ = _fused_mm(xa, _bn_id(d), _ones_row(d), ab)
    p, q = pq[:, : w1.shape[0]], pq[:, w1.shape[0]:]
    qg = q[nbr.reshape(-1)].reshape(n, kk, w1.shape[0])
    v3, st = _edge_lin1(p, qg)
    v = v3.reshape(n * kk, w1.shape[0])
    y2, bn2 = _mlp_raw(v, st, layers, 0.2, n * kk)
    r = _segmax_k(y2.reshape(n, kk, y2.shape[1]))
    return r, bn2


def kernel(x, edge_index, edge_attr, params, k):
    n = x.shape[0]
    kk = 16
    src = edge_index[0]
    dst = edge_index[1]

    ea_max = jax.ops.segment_max(edge_attr, dst, num_segments=n)
    ea_blk = jnp.where(jnp.isfinite(ea_max), ea_max, 0.0)

    sg1 = _mlp_j(_static_agg(x, src, dst, ea_blk, n), params['sg1'], 0.1)
    y2, bn2 = _static_conv(sg1, src, dst, ea_blk, params['sg2'], n)
    y3, bn3 = _static_conv(_apply_act(y2, bn2, 0.1), src, dst, ea_blk,
                           params['sg3'], n)

    dg1 = _dynamic_conv_exact(sg1, params['dg1'], n, kk)
    rd2, bnd2 = _dynamic_conv(dg1, params['dg2'], n, kk)

    bn1 = _bn_id(sg1.shape[1])
    bnd1 = _bn_id(dg1.shape[1])
    cat_raw = jnp.concatenate([sg1, dg1, rd2, y2, y3], axis=1)
    cat_bn = tuple(
        jnp.concatenate([bn1[t], bnd1[t], bnd2[t], bn2[t], bn3[t]], axis=1)
        for t in range(4))
    cat_sl = jnp.concatenate(
        [
            jnp.ones((1, sg1.shape[1] + dg1.shape[1])),
            jnp.full((1, rd2.shape[1]), 0.2),
            jnp.full((1, y2.shape[1] + y3.shape[1]), 0.1),
        ],
        axis=1,
    )

    f1 = params['fuse1']
    yf1, stf1 = _fused_mm(cat_raw, cat_bn, cat_sl, f1[0][0].T)
    bnf1 = _fin_stats(stf1, f1[0][1], f1[0][2], n)

    cat2_raw = jnp.concatenate([yf1, cat_raw], axis=1)
    cat2_bn = tuple(
        jnp.concatenate([bnf1[t], cat_bn[t]], axis=1) for t in range(4))
    cat2_sl = jnp.concatenate(
        [jnp.full((1, yf1.shape[1]), 0.2), cat_sl], axis=1)

    f2 = params['fuse2']
    ya, sta = _fused_mm(cat2_raw, cat2_bn, cat2_sl, f2[0][0].T)
    bna = _fin_stats(sta, f2[0][1], f2[0][2], n)
    yb, stb = _fused_mm(ya, bna, jnp.full((1, ya.shape[1]), 0.2), f2[1][0].T)
    bnb = _fin_stats(stb, f2[1][1], f2[1][2], n)

    out, _ = _fused_mm(yb, bnb, jnp.full((1, yb.shape[1]), 0.2),
                       params['fuse2_out'].T)
    return out


# speedup vs baseline: 1.2395x; 1.0110x over previous
import functools

import jax
import jax.numpy as jnp
from jax import lax
from jax.experimental import pallas as pl
from jax.experimental.pallas import tpu as pltpu, tpu_sc as plsc


def _pick_bm(m, cap=1600):
    for c in (1600, 1000, 800, 400, 320, 200, 160, 100, 80, 40, 16, 8, 4, 2, 1):
        if c <= cap and m % c == 0:
            return c
    return m


def _mm_body(x_ref, m_ref, g_ref, d_ref, b_ref, sl_ref, w_ref, y_ref, st_ref):
    i = pl.program_id(1)
    z = g_ref[...] * (x_ref[...] - m_ref[...]) / d_ref[...] + b_ref[...]
    z = jnp.where(z >= 0, z, z * sl_ref[...])
    y = jnp.dot(z.astype(jnp.bfloat16), w_ref[...],
                preferred_element_type=jnp.float32)
    y_ref[...] = y

    @pl.when(i == 0)
    def _():
        st_ref[...] = jnp.zeros_like(st_ref)

    s = jnp.sum(y, axis=0, keepdims=True)
    ss = jnp.sum(y * y, axis=0, keepdims=True)
    st_ref[...] += jnp.concatenate([s, ss], axis=0)


def _fused_mm(x, bn, sl, wt):
    m, kd = x.shape
    nd = wt.shape[1]
    bm = _pick_bm(m)
    if nd <= 1024:
        bn_blk = nd
    else:
        assert nd % 1024 == 0, nd
        bn_blk = 1024
    grid = (nd // bn_blk, m // bm)
    wt = wt.astype(jnp.bfloat16)
    row = pl.BlockSpec((1, kd), lambda j, i: (0, 0))
    y, st = pl.pallas_call(
        _mm_body,
        grid=grid,
        in_specs=[
            pl.BlockSpec((bm, kd), lambda j, i: (i, 0)),
            row, row, row, row, row,
            pl.BlockSpec((kd, bn_blk), lambda j, i: (0, j)),
        ],
        out_specs=[
            pl.BlockSpec((bm, bn_blk), lambda j, i: (i, j)),
            pl.BlockSpec((2, bn_blk), lambda j, i: (0, j)),
        ],
        out_shape=[
            jax.ShapeDtypeStruct((m, nd), jnp.float32),
            jax.ShapeDtypeStruct((2, nd), jnp.float32),
        ],
    )(x, bn[0], bn[1], bn[2], bn[3], sl, wt)
    return y, st


def _bn_id(kd):
    z = jnp.zeros((1, kd), jnp.float32)
    o = jnp.ones((1, kd), jnp.float32)
    return (z, o, o, z)


def _ones_row(kd):
    return jnp.ones((1, kd), jnp.float32)


def _fin_stats(st, g, b, m):
    mean = st[0] / m
    var = jnp.maximum(st[1] / m - mean * mean, 0.0)
    den = jnp.sqrt(var + 1e-5)
    return (mean[None], g[None], den[None], b[None])


def _fin_stats_exact(y, g, b):
    mean = jnp.mean(y, axis=0)
    den = jnp.sqrt(jnp.var(y, axis=0) + 1e-5)
    return (mean[None], g[None], den[None], b[None])


def _act_body(y_ref, m_ref, g_ref, d_ref, b_ref, o_ref, *, slope):
    z = g_ref[...] * (y_ref[...] - m_ref[...]) / d_ref[...] + b_ref[...]
    o_ref[...] = jnp.where(z >= 0, z, z * slope)


def _apply_act(y, bn, slope):
    m, nd = y.shape
    bm = _pick_bm(m)
    row = pl.BlockSpec((1, nd), lambda i: (0, 0))
    return pl.pallas_call(
        functools.partial(_act_body, slope=slope),
        grid=(m // bm,),
        in_specs=[pl.BlockSpec((bm, nd), lambda i: (i, 0)),
                  row, row, row, row],
        out_specs=pl.BlockSpec((bm, nd), lambda i: (i, 0)),
        out_shape=jax.ShapeDtypeStruct((m, nd), jnp.float32),
    )(y, bn[0], bn[1], bn[2], bn[3])


def _addstats_body(p_ref, q_ref, v_ref, st_ref):
    i = pl.program_id(0)
    v = q_ref[...] + p_ref[...][:, None, :]
    v_ref[...] = v

    @pl.when(i == 0)
    def _():
        st_ref[...] = jnp.zeros_like(st_ref)

    s = jnp.sum(jnp.sum(v, axis=1), axis=0, keepdims=True)
    ss = jnp.sum(jnp.sum(v * v, axis=1), axis=0, keepdims=True)
    st_ref[...] += jnp.concatenate([s, ss], axis=0)


def _edge_lin1(p, qg):
    n, d = p.shape
    kk = qg.shape[1]
    bm = _pick_bm(n, cap=200)
    v, st = pl.pallas_call(
        _addstats_body,
        grid=(n // bm,),
        in_specs=[
            pl.BlockSpec((bm, d), lambda i: (i, 0)),
            pl.BlockSpec((bm, kk, d), lambda i: (i, 0, 0)),
        ],
        out_specs=[
            pl.BlockSpec((bm, kk, d), lambda i: (i, 0, 0)),
            pl.BlockSpec((2, d), lambda i: (0, 0)),
        ],
        out_shape=[
            jax.ShapeDtypeStruct((n, kk, d), jnp.float32),
            jax.ShapeDtypeStruct((2, d), jnp.float32),
        ],
    )(p, qg)
    return v, st


def _segmax_body(y_ref, o_ref):
    o_ref[...] = jnp.max(y_ref[...], axis=1)


def _segmax_k(y3):
    n, kk, d = y3.shape
    bm = _pick_bm(n, cap=200)
    return pl.pallas_call(
        _segmax_body,
        grid=(n // bm,),
        in_specs=[pl.BlockSpec((bm, kk, d), lambda i: (i, 0, 0))],
        out_specs=pl.BlockSpec((bm, d), lambda i: (i, 0)),
        out_shape=jax.ShapeDtypeStruct((n, d), jnp.float32),
    )(y3)


def _leaky_j(h, s):
    return jnp.where(h >= 0, h, s * h)


def _bn_j(h, g, b):
    m = jnp.mean(h, axis=0)
    v = jnp.var(h, axis=0)
    return g * (h - m) / jnp.sqrt(v + 1e-5) + b


def _mlp_j(h, layers, slope):
    for (w, g, b) in layers:
        h = _leaky_j(_bn_j(h @ w.T, g, b), slope)
    return h


def _mlp_raw(y, st, layers, slope, m, exact=False):
    if exact:
        bn = _fin_stats_exact(y, layers[0][1], layers[0][2])
    else:
        bn = _fin_stats(st, layers[0][1], layers[0][2], m)
    for li in range(1, len(layers)):
        wt = layers[li][0].T
        y, st = _fused_mm(y, bn, jnp.full((1, y.shape[1]), slope), wt)
        if exact:
            bn = _fin_stats_exact(y, layers[li][1], layers[li][2])
        else:
            bn = _fin_stats(st, layers[li][1], layers[li][2], m)
    return y, bn


def _static_agg(xa, src, dst, ea_blk, n):
    smax = jax.ops.segment_max(xa[src], dst, num_segments=n)
    mask = jnp.isfinite(smax[:, :1])
    return jnp.concatenate(
        [
            jnp.where(mask, xa, 0.0),
            jnp.where(mask, smax - xa, 0.0),
            ea_blk,
        ],
        axis=1,
    )


def _static_conv(xa, src, dst, ea_blk, layers, n):
    agg = _static_agg(xa, src, dst, ea_blk, n)
    kd = agg.shape[1]
    y, st = _fused_mm(agg, _bn_id(kd), _ones_row(kd), layers[0][0].T)
    return _mlp_raw(y, st, layers, 0.1, n)


def _sc_gather(table, idx):
    d = table.shape[1]
    b = idx.shape[0]
    info = plsc.get_sparse_core_info()
    nw = info.num_cores * info.num_subcores
    assert b % nw == 0, (b, nw)
    bpw = b // nw
    chunk = max(c for c in range(8, 129, 8) if bpw % c == 0)
    nchunks = bpw // chunk
    mesh = plsc.VectorSubcoreMesh(core_axis_name="c", subcore_axis_name="s")

    @functools.partial(
        pl.kernel,
        mesh=mesh,
        out_type=jax.ShapeDtypeStruct((b, d), jnp.float32),
        scratch_types=[
            pltpu.VMEM((chunk,), jnp.int32),
            pltpu.VMEM((chunk, d), jnp.float32),
            pltpu.SemaphoreType.DMA,
        ],
    )
    def k(table_hbm, idx_hbm, out_hbm, idx_v, rows_v, sem):
        wid = lax.axis_index("s") * info.num_cores + lax.axis_index("c")
        base = wid * bpw

        def body(c, _):
            off = base + c * chunk
            pltpu.sync_copy(idx_hbm.at[pl.ds(off, chunk)], idx_v)
            pltpu.async_copy(table_hbm.at[idx_v], rows_v, sem).wait()
            pltpu.sync_copy(rows_v, out_hbm.at[pl.ds(off, chunk)])
            return ()

        lax.fori_loop(0, nchunks, body, ())

    return k(table, idx)


def _knn(xa, n, kk):
    xn = xa / (jnp.linalg.norm(xa, axis=1, keepdims=True) + 1e-8)
    d = xn.shape[1]
    npad = ((n + 1023) // 1024) * 1024
    wt = jnp.pad(xn.T, ((0, 0), (0, npad - n)))
    sim, _ = _fused_mm(xn, _bn_id(d), _ones_row(d), wt)
    _, nbr = lax.top_k(sim[:, :n], kk)
    return nbr


def _dynamic_conv_exact(xa, layers, n, kk):
    nbr = _knn(xa, n, kk)
    x_j = _sc_gather(xa, nbr.reshape(-1))
    x_i = jnp.repeat(xa, kk, axis=0)
    msg = jnp.concatenate([x_i, x_j - x_i], axis=1)
    msg = _mlp_j(msg, layers, 0.2)
    r = jnp.max(msg.reshape(n, kk, msg.shape[1]), axis=1)
    return jnp.where(jnp.isfinite(r), r, 0.0)


def _dynamic_conv(xa, layers, n, kk):
    nbr = _knn(xa, n, kk)
    d = xa.shape[1]
    w1 = layers[0][0]
    h = w1.shape[1] // 2
    w1a, w1b = w1[:, :h], w1[:, h:]
    ab = jnp.concatenate([(w1a - w1b).T, w1b.T], axis=1)
    pq, _ = _fused_mm(xa, _bn_id(d), _ones_row(d), ab)
    p, q = pq[:, : w1.shape[0]], pq[:, w1.shape[0]:]
    qg = _sc_gather(q, nbr.reshape(-1)).reshape(n, kk, w1.shape[0])
    v3, st = _edge_lin1(p, qg)
    v = v3.reshape(n * kk, w1.shape[0])
    y2, bn2 = _mlp_raw(v, st, layers, 0.2, n * kk)
    r = _segmax_k(y2.reshape(n, kk, y2.shape[1]))
    return r, bn2


def kernel(x, edge_index, edge_attr, params, k):
    n = x.shape[0]
    kk = 16
    src = edge_index[0]
    dst = edge_index[1]

    ea_max = jax.ops.segment_max(edge_attr, dst, num_segments=n)
    ea_blk = jnp.where(jnp.isfinite(ea_max), ea_max, 0.0)

    sg1 = _mlp_j(_static_agg(x, src, dst, ea_blk, n), params['sg1'], 0.1)
    y2, bn2 = _static_conv(sg1, src, dst, ea_blk, params['sg2'], n)
    y3, bn3 = _static_conv(_apply_act(y2, bn2, 0.1), src, dst, ea_blk,
                           params['sg3'], n)

    dg1 = _dynamic_conv_exact(sg1, params['dg1'], n, kk)
    rd2, bnd2 = _dynamic_conv(dg1, params['dg2'], n, kk)

    bn1 = _bn_id(sg1.shape[1])
    bnd1 = _bn_id(dg1.shape[1])
    cat_raw = jnp.concatenate([sg1, dg1, rd2, y2, y3], axis=1)
    cat_bn = tuple(
        jnp.concatenate([bn1[t], bnd1[t], bnd2[t], bn2[t], bn3[t]], axis=1)
        for t in range(4))
    cat_sl = jnp.concatenate(
        [
            jnp.ones((1, sg1.shape[1] + dg1.shape[1])),
            jnp.full((1, rd2.shape[1]), 0.2),
            jnp.full((1, y2.shape[1] + y3.shape[1]), 0.1),
        ],
        axis=1,
    )

    f1 = params['fuse1']
    yf1, stf1 = _fused_mm(cat_raw, cat_bn, cat_sl, f1[0][0].T)
    bnf1 = _fin_stats(stf1, f1[0][1], f1[0][2], n)

    cat2_raw = jnp.concatenate([yf1, cat_raw], axis=1)
    cat2_bn = tuple(
        jnp.concatenate([bnf1[t], cat_bn[t]], axis=1) for t in range(4))
    cat2_sl = jnp.concatenate(
        [jnp.full((1, yf1.shape[1]), 0.2), cat_sl], axis=1)

    f2 = params['fuse2']
    ya, sta = _fused_mm(cat2_raw, cat2_bn, cat2_sl, f2[0][0].T)
    bna = _fin_stats(sta, f2[0][1], f2[0][2], n)
    yb, stb = _fused_mm(ya, bna, jnp.full((1, ya.shape[1]), 0.2), f2[1][0].T)
    bnb = _fin_stats(stb, f2[1][1], f2[1][2], n)

    out, _ = _fused_mm(yb, bnb, jnp.full((1, yb.shape[1]), 0.2),
                       params['fuse2_out'].T)
    return out
